# baseline (speedup 1.0000x reference)
import jax
import jax.numpy as jnp
from jax import lax
from jax.experimental import pallas as pl
from jax.experimental.pallas import tpu as pltpu
from jax.experimental.pallas import tpu_sc as plsc
import functools

NUM_RADIAL = 8
NUM_SPHERICAL = 8
NB64 = NUM_RADIAL * NUM_SPHERICAL
NC = 2
NS = 16
NW = NC * NS

CHUNK = 640
IDX_ROWS = CHUNK // 128

R_COEFF = -0.5 * (NUM_RADIAL - 1) ** 2
S_COEFF = -0.5 * ((NUM_SPHERICAL - 1) / 2.0) ** 2


def _make_sc_gather(T):
    n_chunks = T // CHUNK
    mesh = plsc.VectorSubcoreMesh(
        core_axis_name="c", subcore_axis_name="s",
        num_cores=NC, num_subcores=NS)

    @functools.partial(
        pl.kernel,
        out_type=jax.ShapeDtypeStruct((T,), jnp.float32),
        mesh=mesh,
        compiler_params=pltpu.CompilerParams(use_tc_tiling_on_sc=False,
                                             needs_layout_passes=False),
        scratch_types=[
            pltpu.VMEM((2 * IDX_ROWS, 128), jnp.int32),
            pltpu.VMEM((2 * CHUNK,), jnp.float32),
            pltpu.SemaphoreType.DMA,
            pltpu.SemaphoreType.DMA,
            pltpu.SemaphoreType.DMA,
        ],
    )
    def sc_kernel(d_hbm, id3_hbm, out_hbm, idx_v, dg_v, sem_i, sem_g, sem_o):
        wid = lax.axis_index("s") * NC + lax.axis_index("c")
        nj = (n_chunks - wid + NW - 1) // NW

        def cid_of(j):
            return wid + j * NW

        def issue_in(j, b):
            pltpu.async_copy(
                id3_hbm.at[pl.ds(cid_of(j) * IDX_ROWS, IDX_ROWS)],
                idx_v.at[pl.ds(b * IDX_ROWS, IDX_ROWS)], sem_i)

        def wait_in(b):
            pltpu.make_async_copy(
                id3_hbm.at[pl.ds(0, IDX_ROWS)],
                idx_v.at[pl.ds(b * IDX_ROWS, IDX_ROWS)], sem_i).wait()

        def issue_gather(b):
            for k in range(IDX_ROWS):
                pltpu.async_copy(
                    d_hbm.at[idx_v.at[b * IDX_ROWS + k]],
                    dg_v.at[pl.ds(b * CHUNK + k * 128, 128)], sem_g)

        def wait_gather(b):
            for k in range(IDX_ROWS):
                pltpu.make_async_copy(
                    d_hbm.at[idx_v.at[b * IDX_ROWS + k]],
                    dg_v.at[pl.ds(b * CHUNK + k * 128, 128)], sem_g).wait()

        def out_desc(j, b):
            return pltpu.make_async_copy(
                dg_v.at[pl.ds(b * CHUNK, CHUNK)],
                out_hbm.at[pl.ds(cid_of(j) * CHUNK, CHUNK)], sem_o)

        issue_in(0, 0)
        wait_in(0)
        issue_gather(0)
        issue_in(1, 1)

        @pl.loop(0, nj)
        def _chunk(j):
            b = lax.rem(j, 2)
            nb = 1 - b

            @pl.when(j + 1 < nj)
            def _():
                wait_in(nb)

                @pl.when(j >= 1)
                def _():
                    out_desc(j - 1, nb).wait()

                issue_gather(nb)

            wait_gather(b)
            pltpu.async_copy(dg_v.at[pl.ds(b * CHUNK, CHUNK)],
                             out_hbm.at[pl.ds(cid_of(j) * CHUNK, CHUNK)],
                             sem_o)

            @pl.when(j + 2 < nj)
            def _():
                issue_in(j + 2, b)

        @pl.when(nj >= 2)
        def _():
            out_desc(nj - 2, lax.rem(nj - 2, 2)).wait()

        out_desc(nj - 1, lax.rem(nj - 1, 2)).wait()

    return sc_kernel


ROWS = 50


def _tc_outer(dg, cosphi, T):
    rows_total = T // 128
    nsteps = rows_total // ROWS

    def body(c_ref, d_ref, p_ref, out_ref):
        i = pl.program_id(0)
        sub = lax.broadcasted_iota(
            jnp.int32, (NUM_RADIAL, 1), 0).astype(jnp.float32)
        ro = sub / (NUM_RADIAL - 1)
        so = sub * (2.0 / (NUM_SPHERICAL - 1)) - 1.0
        pmat = p_ref[...]
        def permute(ref):
            return lax.dot_general(
                ref[pl.ds(i * ROWS, ROWS), :], pmat,
                (((1,), (0,)), ((), ())),
                precision=lax.Precision.HIGHEST,
                preferred_element_type=jnp.float32)

        cp = permute(c_ref)
        dp = permute(d_ref)
        for r in range(ROWS):
            d8 = jnp.broadcast_to(dp[r:r + 1, :], (NUM_RADIAL, 128))
            c8 = jnp.broadcast_to(cp[r:r + 1, :], (NUM_SPHERICAL, 128))
            dd = d8 - ro
            cc = c8 - so
            rbf = jnp.exp(R_COEFF * dd * dd)
            cbf = jnp.exp(S_COEFF * cc * cc)
            prod = jnp.concatenate(
                [rbf * jnp.broadcast_to(cbf[s:s + 1, :], (NUM_RADIAL, 128))
                 for s in range(NUM_SPHERICAL)], axis=0)
            prod_t = prod.T
            out_ref[r * 64:(r + 1) * 64, :] = jnp.concatenate(
                [prod_t[:64, :], prod_t[64:, :]], axis=1)

    import numpy as np
    dst = np.arange(128)
    src = np.where(dst < 64, 2 * dst, 2 * (dst - 64) + 1)
    pmat = np.zeros((128, 128), np.float32)
    pmat[src, dst] = 1.0

    whole = pl.BlockSpec((rows_total, 128), lambda i: (0, 0))
    return pl.pallas_call(
        body,
        grid=(nsteps,),
        in_specs=[whole, whole, pl.BlockSpec((128, 128), lambda i: (0, 0))],
        out_specs=pl.BlockSpec((ROWS * 64, 2 * NB64), lambda i: (i, 0)),
        out_shape=jax.ShapeDtypeStruct((T // 2, 2 * NB64), jnp.float32),
    )(cosphi.reshape(rows_total, 128), dg.reshape(rows_total, 128),
      jnp.asarray(pmat))


def kernel(D_ca, cosphi_cab, id3_ca):
    T = cosphi_cab.shape[0]
    id3_2d = jnp.asarray(id3_ca, jnp.int32).reshape(T // 128, 128)
    dg = _make_sc_gather(T)(jnp.asarray(D_ca, jnp.float32), id3_2d)
    out = _tc_outer(dg, jnp.asarray(cosphi_cab, jnp.float32), T)
    return (out.reshape(T, NB64),)


# speedup vs baseline: 1.5731x; 1.5731x over previous
import jax
import jax.numpy as jnp
from jax import lax
from jax.experimental import pallas as pl
from jax.experimental.pallas import tpu as pltpu
from jax.experimental.pallas import tpu_sc as plsc
import functools

NUM_RADIAL = 8
NUM_SPHERICAL = 8
NB64 = NUM_RADIAL * NUM_SPHERICAL
NC = 2
NS = 16
NW = NC * NS

CHUNK = 640
IDX_ROWS = CHUNK // 128

R_COEFF = -0.5 * (NUM_RADIAL - 1) ** 2
S_COEFF = -0.5 * ((NUM_SPHERICAL - 1) / 2.0) ** 2


def _make_sc_gather(T):
    n_chunks = T // CHUNK
    mesh = plsc.VectorSubcoreMesh(
        core_axis_name="c", subcore_axis_name="s",
        num_cores=NC, num_subcores=NS)

    @functools.partial(
        pl.kernel,
        out_type=jax.ShapeDtypeStruct((T,), jnp.float32),
        mesh=mesh,
        compiler_params=pltpu.CompilerParams(use_tc_tiling_on_sc=False,
                                             needs_layout_passes=False),
        scratch_types=[
            pltpu.VMEM((2 * IDX_ROWS, 128), jnp.int32),
            pltpu.VMEM((2 * CHUNK,), jnp.float32),
            pltpu.SemaphoreType.DMA,
            pltpu.SemaphoreType.DMA,
            pltpu.SemaphoreType.DMA,
        ],
    )
    def sc_kernel(d_hbm, id3_hbm, out_hbm, idx_v, dg_v, sem_i, sem_g, sem_o):
        wid = lax.axis_index("s") * NC + lax.axis_index("c")
        nj = (n_chunks - wid + NW - 1) // NW

        def cid_of(j):
            return wid + j * NW

        def issue_in(j, b):
            pltpu.async_copy(
                id3_hbm.at[pl.ds(cid_of(j) * IDX_ROWS, IDX_ROWS)],
                idx_v.at[pl.ds(b * IDX_ROWS, IDX_ROWS)], sem_i)

        def wait_in(b):
            pltpu.make_async_copy(
                id3_hbm.at[pl.ds(0, IDX_ROWS)],
                idx_v.at[pl.ds(b * IDX_ROWS, IDX_ROWS)], sem_i).wait()

        def issue_gather(b):
            for k in range(IDX_ROWS):
                pltpu.async_copy(
                    d_hbm.at[idx_v.at[b * IDX_ROWS + k]],
                    dg_v.at[pl.ds(b * CHUNK + k * 128, 128)], sem_g)

        def wait_gather(b):
            for k in range(IDX_ROWS):
                pltpu.make_async_copy(
                    d_hbm.at[idx_v.at[b * IDX_ROWS + k]],
                    dg_v.at[pl.ds(b * CHUNK + k * 128, 128)], sem_g).wait()

        def out_desc(j, b):
            return pltpu.make_async_copy(
                dg_v.at[pl.ds(b * CHUNK, CHUNK)],
                out_hbm.at[pl.ds(cid_of(j) * CHUNK, CHUNK)], sem_o)

        issue_in(0, 0)
        wait_in(0)
        issue_gather(0)
        issue_in(1, 1)

        @pl.loop(0, nj)
        def _chunk(j):
            b = lax.rem(j, 2)
            nb = 1 - b

            @pl.when(j + 1 < nj)
            def _():
                wait_in(nb)

                @pl.when(j >= 1)
                def _():
                    out_desc(j - 1, nb).wait()

                issue_gather(nb)

            wait_gather(b)
            pltpu.async_copy(dg_v.at[pl.ds(b * CHUNK, CHUNK)],
                             out_hbm.at[pl.ds(cid_of(j) * CHUNK, CHUNK)],
                             sem_o)

            @pl.when(j + 2 < nj)
            def _():
                issue_in(j + 2, b)

        @pl.when(nj >= 2)
        def _():
            out_desc(nj - 2, lax.rem(nj - 2, 2)).wait()

        out_desc(nj - 1, lax.rem(nj - 1, 2)).wait()

    return sc_kernel


ROWS = 125


def _tc_outer(dg, cosphi, T):
    rows_total = T // 128
    nsteps = rows_total // ROWS

    def body(c_ref, d_ref, out_ref):
        i = pl.program_id(0)
        sub = lax.broadcasted_iota(
            jnp.int32, (NUM_RADIAL, 1), 0).astype(jnp.float32)
        ro = sub / (NUM_RADIAL - 1)
        so = sub * (2.0 / (NUM_SPHERICAL - 1)) - 1.0
        for r in range(ROWS):
            row = i * ROWS + r
            d8 = jnp.broadcast_to(d_ref[pl.ds(row, 1), :], (NUM_RADIAL, 128))
            c8 = jnp.broadcast_to(c_ref[pl.ds(row, 1), :], (NUM_SPHERICAL, 128))
            dd = d8 - ro
            cc = c8 - so
            rbf = jnp.exp(R_COEFF * dd * dd)
            cbf = jnp.exp(S_COEFF * cc * cc)
            prod = jnp.concatenate(
                [rbf * jnp.broadcast_to(cbf[s:s + 1, :], (NUM_RADIAL, 128))
                 for s in range(NUM_SPHERICAL)], axis=0)
            out_ref[r * 128:(r + 1) * 128, :] = prod.T

    whole = pl.BlockSpec((rows_total, 128), lambda i: (0, 0))
    return pl.pallas_call(
        body,
        grid=(nsteps,),
        in_specs=[whole, whole],
        out_specs=pl.BlockSpec((ROWS * 128, NB64), lambda i: (i, 0)),
        out_shape=jax.ShapeDtypeStruct((T, NB64), jnp.float32),
    )(cosphi.reshape(rows_total, 128), dg.reshape(rows_total, 128))


def kernel(D_ca, cosphi_cab, id3_ca):
    T = cosphi_cab.shape[0]
    id3_2d = jnp.asarray(id3_ca, jnp.int32).reshape(T // 128, 128)
    dg = _make_sc_gather(T)(jnp.asarray(D_ca, jnp.float32), id3_2d)
    return (_tc_outer(dg, jnp.asarray(cosphi_cab, jnp.float32), T),)


# speedup vs baseline: 1.5901x; 1.0108x over previous
import jax
import jax.numpy as jnp
from jax import lax
from jax.experimental import pallas as pl
from jax.experimental.pallas import tpu as pltpu
from jax.experimental.pallas import tpu_sc as plsc
import functools

NUM_RADIAL = 8
NUM_SPHERICAL = 8
NB64 = NUM_RADIAL * NUM_SPHERICAL
NC = 2
NS = 16
NW = NC * NS

CHUNK = 640
IDX_ROWS = CHUNK // 128

R_COEFF = -0.5 * (NUM_RADIAL - 1) ** 2
S_COEFF = -0.5 * ((NUM_SPHERICAL - 1) / 2.0) ** 2


def _make_sc_gather(T):
    n_chunks = T // CHUNK
    mesh = plsc.VectorSubcoreMesh(
        core_axis_name="c", subcore_axis_name="s",
        num_cores=NC, num_subcores=NS)

    @functools.partial(
        pl.kernel,
        out_type=jax.ShapeDtypeStruct((T,), jnp.float32),
        mesh=mesh,
        compiler_params=pltpu.CompilerParams(use_tc_tiling_on_sc=False,
                                             needs_layout_passes=False),
        scratch_types=[
            pltpu.VMEM((2 * IDX_ROWS, 128), jnp.int32),
            pltpu.VMEM((2 * CHUNK,), jnp.float32),
            pltpu.SemaphoreType.DMA,
            pltpu.SemaphoreType.DMA,
            pltpu.SemaphoreType.DMA,
        ],
    )
    def sc_kernel(d_hbm, id3_hbm, out_hbm, idx_v, dg_v, sem_i, sem_g, sem_o):
        wid = lax.axis_index("s") * NC + lax.axis_index("c")
        nj = (n_chunks - wid + NW - 1) // NW

        def cid_of(j):
            return wid + j * NW

        def issue_in(j, b):
            pltpu.async_copy(
                id3_hbm.at[pl.ds(cid_of(j) * IDX_ROWS, IDX_ROWS)],
                idx_v.at[pl.ds(b * IDX_ROWS, IDX_ROWS)], sem_i)

        def wait_in(b):
            pltpu.make_async_copy(
                id3_hbm.at[pl.ds(0, IDX_ROWS)],
                idx_v.at[pl.ds(b * IDX_ROWS, IDX_ROWS)], sem_i).wait()

        def issue_gather(b):
            for k in range(IDX_ROWS):
                pltpu.async_copy(
                    d_hbm.at[idx_v.at[b * IDX_ROWS + k]],
                    dg_v.at[pl.ds(b * CHUNK + k * 128, 128)], sem_g)

        def wait_gather(b):
            for k in range(IDX_ROWS):
                pltpu.make_async_copy(
                    d_hbm.at[idx_v.at[b * IDX_ROWS + k]],
                    dg_v.at[pl.ds(b * CHUNK + k * 128, 128)], sem_g).wait()

        def out_desc(j, b):
            return pltpu.make_async_copy(
                dg_v.at[pl.ds(b * CHUNK, CHUNK)],
                out_hbm.at[pl.ds(cid_of(j) * CHUNK, CHUNK)], sem_o)

        issue_in(0, 0)
        wait_in(0)
        issue_gather(0)
        issue_in(1, 1)

        @pl.loop(0, nj)
        def _chunk(j):
            b = lax.rem(j, 2)
            nb = 1 - b

            @pl.when(j + 1 < nj)
            def _():
                wait_in(nb)

                @pl.when(j >= 1)
                def _():
                    out_desc(j - 1, nb).wait()

                issue_gather(nb)

            wait_gather(b)
            pltpu.async_copy(dg_v.at[pl.ds(b * CHUNK, CHUNK)],
                             out_hbm.at[pl.ds(cid_of(j) * CHUNK, CHUNK)],
                             sem_o)

            @pl.when(j + 2 < nj)
            def _():
                issue_in(j + 2, b)

        @pl.when(nj >= 2)
        def _():
            out_desc(nj - 2, lax.rem(nj - 2, 2)).wait()

        out_desc(nj - 1, lax.rem(nj - 1, 2)).wait()

    return sc_kernel


ROWS = 250


def _tc_outer(dg, cosphi, T):
    rows_total = T // 128
    nsteps = rows_total // ROWS

    def body(c_ref, d_ref, out_ref):
        i = pl.program_id(0)
        sub = lax.broadcasted_iota(
            jnp.int32, (NUM_RADIAL, 1), 0).astype(jnp.float32)
        ro = sub / (NUM_RADIAL - 1)
        so = sub * (2.0 / (NUM_SPHERICAL - 1)) - 1.0
        for r in range(ROWS):
            row = i * ROWS + r
            d8 = jnp.broadcast_to(d_ref[pl.ds(row, 1), :], (NUM_RADIAL, 128))
            c8 = jnp.broadcast_to(c_ref[pl.ds(row, 1), :], (NUM_SPHERICAL, 128))
            dd = d8 - ro
            cc = c8 - so
            rbf = jnp.exp(R_COEFF * dd * dd)
            cbf = jnp.exp(S_COEFF * cc * cc)
            prod = jnp.concatenate(
                [rbf * jnp.broadcast_to(cbf[s:s + 1, :], (NUM_RADIAL, 128))
                 for s in range(NUM_SPHERICAL)], axis=0)
            out_ref[r * 128:(r + 1) * 128, :] = prod.T

    whole = pl.BlockSpec((rows_total, 128), lambda i: (0, 0))
    return pl.pallas_call(
        body,
        grid=(nsteps,),
        in_specs=[whole, whole],
        out_specs=pl.BlockSpec((ROWS * 128, NB64), lambda i: (i, 0)),
        out_shape=jax.ShapeDtypeStruct((T, NB64), jnp.float32),
    )(cosphi.reshape(rows_total, 128), dg.reshape(rows_total, 128))


def kernel(D_ca, cosphi_cab, id3_ca):
    T = cosphi_cab.shape[0]
    id3_2d = jnp.asarray(id3_ca, jnp.int32).reshape(T // 128, 128)
    dg = _make_sc_gather(T)(jnp.asarray(D_ca, jnp.float32), id3_2d)
    return (_tc_outer(dg, jnp.asarray(cosphi_cab, jnp.float32), T),)


# speedup vs baseline: 1.6250x; 1.0219x over previous
import jax
import jax.numpy as jnp
from jax import lax
from jax.experimental import pallas as pl
from jax.experimental.pallas import tpu as pltpu
from jax.experimental.pallas import tpu_sc as plsc
import functools

NUM_RADIAL = 8
NUM_SPHERICAL = 8
NB64 = NUM_RADIAL * NUM_SPHERICAL
NC = 2
NS = 16
NW = NC * NS

CHUNK = 640
IDX_ROWS = CHUNK // 128

R_COEFF = -0.5 * (NUM_RADIAL - 1) ** 2
S_COEFF = -0.5 * ((NUM_SPHERICAL - 1) / 2.0) ** 2


def _make_sc_gather(T):
    n_chunks = T // CHUNK
    mesh = plsc.VectorSubcoreMesh(
        core_axis_name="c", subcore_axis_name="s",
        num_cores=NC, num_subcores=NS)

    @functools.partial(
        pl.kernel,
        out_type=jax.ShapeDtypeStruct((T,), jnp.float32),
        mesh=mesh,
        compiler_params=pltpu.CompilerParams(use_tc_tiling_on_sc=False,
                                             needs_layout_passes=False),
        scratch_types=[
            pltpu.VMEM((2 * IDX_ROWS, 128), jnp.int32),
            pltpu.VMEM((2 * CHUNK,), jnp.float32),
            pltpu.SemaphoreType.DMA,
            pltpu.SemaphoreType.DMA,
            pltpu.SemaphoreType.DMA,
        ],
    )
    def sc_kernel(d_hbm, id3_hbm, out_hbm, idx_v, dg_v, sem_i, sem_g, sem_o):
        wid = lax.axis_index("s") * NC + lax.axis_index("c")
        nj = (n_chunks - wid + NW - 1) // NW

        def cid_of(j):
            return wid + j * NW

        def issue_in(j, b):
            pltpu.async_copy(
                id3_hbm.at[pl.ds(cid_of(j) * IDX_ROWS, IDX_ROWS)],
                idx_v.at[pl.ds(b * IDX_ROWS, IDX_ROWS)], sem_i)

        def wait_in(b):
            pltpu.make_async_copy(
                id3_hbm.at[pl.ds(0, IDX_ROWS)],
                idx_v.at[pl.ds(b * IDX_ROWS, IDX_ROWS)], sem_i).wait()

        def issue_gather(b):
            for k in range(IDX_ROWS):
                pltpu.async_copy(
                    d_hbm.at[idx_v.at[b * IDX_ROWS + k]],
                    dg_v.at[pl.ds(b * CHUNK + k * 128, 128)], sem_g)

        def wait_gather(b):
            for k in range(IDX_ROWS):
                pltpu.make_async_copy(
                    d_hbm.at[idx_v.at[b * IDX_ROWS + k]],
                    dg_v.at[pl.ds(b * CHUNK + k * 128, 128)], sem_g).wait()

        def out_desc(j, b):
            return pltpu.make_async_copy(
                dg_v.at[pl.ds(b * CHUNK, CHUNK)],
                out_hbm.at[pl.ds(cid_of(j) * CHUNK, CHUNK)], sem_o)

        issue_in(0, 0)
        wait_in(0)
        issue_gather(0)
        issue_in(1, 1)

        @pl.loop(0, nj)
        def _chunk(j):
            b = lax.rem(j, 2)
            nb = 1 - b

            @pl.when(j + 1 < nj)
            def _():
                wait_in(nb)

                @pl.when(j >= 1)
                def _():
                    out_desc(j - 1, nb).wait()

                issue_gather(nb)

            wait_gather(b)
            pltpu.async_copy(dg_v.at[pl.ds(b * CHUNK, CHUNK)],
                             out_hbm.at[pl.ds(cid_of(j) * CHUNK, CHUNK)],
                             sem_o)

            @pl.when(j + 2 < nj)
            def _():
                issue_in(j + 2, b)

        @pl.when(nj >= 2)
        def _():
            out_desc(nj - 2, lax.rem(nj - 2, 2)).wait()

        out_desc(nj - 1, lax.rem(nj - 1, 2)).wait()

    return sc_kernel


ROWS = 125


def _tc_outer(dg, cosphi, T, Ttot, base_step=0, carry=None):
    rows_total = T // 128
    nsteps = rows_total // ROWS

    def body(c_ref, d_ref, *rest):
        out_ref = rest[-1]
        i = pl.program_id(0)
        sub = lax.broadcasted_iota(
            jnp.int32, (NUM_RADIAL, 1), 0).astype(jnp.float32)
        ro = sub / (NUM_RADIAL - 1)
        so = sub * (2.0 / (NUM_SPHERICAL - 1)) - 1.0
        for r in range(ROWS):
            row = i * ROWS + r
            d8 = jnp.broadcast_to(d_ref[pl.ds(row, 1), :], (NUM_RADIAL, 128))
            c8 = jnp.broadcast_to(c_ref[pl.ds(row, 1), :], (NUM_SPHERICAL, 128))
            dd = d8 - ro
            cc = c8 - so
            rbf = jnp.exp(R_COEFF * dd * dd)
            cbf = jnp.exp(S_COEFF * cc * cc)
            prod = jnp.concatenate(
                [rbf * jnp.broadcast_to(cbf[s:s + 1, :], (NUM_RADIAL, 128))
                 for s in range(NUM_SPHERICAL)], axis=0)
            out_ref[r * 128:(r + 1) * 128, :] = prod.T

    whole = pl.BlockSpec((rows_total, 128), lambda i: (0, 0))
    in_specs = [whole, whole]
    operands = [cosphi.reshape(rows_total, 128), dg.reshape(rows_total, 128)]
    kwargs = {}
    if carry is not None:
        in_specs.append(pl.BlockSpec(memory_space=pl.ANY))
        operands.append(carry)
        kwargs["input_output_aliases"] = {2: 0}
    return pl.pallas_call(
        body,
        grid=(nsteps,),
        in_specs=in_specs,
        out_specs=pl.BlockSpec((ROWS * 128, NB64),
                               lambda i: (i + base_step, 0)),
        out_shape=jax.ShapeDtypeStruct((Ttot, NB64), jnp.float32),
        **kwargs,
    )(*operands)


def kernel(D_ca, cosphi_cab, id3_ca):
    T = cosphi_cab.shape[0]
    Th = T // 2
    D = jnp.asarray(D_ca, jnp.float32)
    id3 = jnp.asarray(id3_ca, jnp.int32)
    cos = jnp.asarray(cosphi_cab, jnp.float32)
    sc = _make_sc_gather(Th)
    dg1 = sc(D, id3[:Th].reshape(Th // 128, 128))
    dg2 = sc(D, id3[Th:].reshape(Th // 128, 128))
    half_steps = (Th // 128) // ROWS
    out1 = _tc_outer(dg1, cos[:Th], Th, T, 0)
    out2 = _tc_outer(dg2, cos[Th:], Th, T, half_steps, carry=out1)
    return (out2,)
